# Initial kernel scaffold; baseline (speedup 1.0000x reference)
import jax
import jax.numpy as jnp
from jax.experimental import pallas as pl


def kernel(x, edge_index, edge_attr, W_f, a_src_f, a_dst_f, W_e_f, a_e_f, b_f, W_b, a_src_b, a_dst_b, W_e_b, a_e_b, b_b, W_c, b_c, gamma, beta):
    raise NotImplementedError("write your pallas kernel here")


# speedup vs baseline: 1.3860x; 1.3860x over previous
import jax
import jax.numpy as jnp
from jax.experimental import pallas as pl
from jax.experimental.pallas import tpu as pltpu

N = 10000
E = 320000
D = 128
C = 64


def _leaky(v):
    return jnp.where(v > 0, v, 0.2 * v)


def _combine_body(f_ref, b_ref, Wc_ref, bc_ref, gamma_ref, beta_ref, out_ref):
    comb = jnp.concatenate([f_ref[...], b_ref[...]], axis=1)
    z = jax.lax.dot_general(comb, Wc_ref[...], (((1,), (1,)), ((), ())),
                            preferred_element_type=jnp.float32)
    z = z + bc_ref[...][None, :]
    mu = jnp.mean(z, axis=0, keepdims=True)
    var = jnp.mean((z - mu) ** 2, axis=0, keepdims=True)
    out = gamma_ref[...][None, :] * (z - mu) * jax.lax.rsqrt(var + 1e-5) + beta_ref[...][None, :]
    out_ref[...] = jnp.maximum(out, 0.0)


def _combine(f_out, b_out, W_c, b_c, gamma, beta):
    return pl.pallas_call(
        _combine_body,
        out_shape=jax.ShapeDtypeStruct((N, 128), jnp.float32),
    )(f_out, b_out, W_c, b_c, gamma, beta)


def _gat_dir(x, src, dst, ea, mask, ea_mean, W, a_src, a_dst, W_e, a_e, b):
    n = x.shape[0]
    h = x @ W
    s_src = h @ a_src
    s_dst = h @ a_dst
    w_e = W_e @ a_e
    e = ea @ w_e
    alpha = s_src[src] + s_dst[dst] + e
    alpha = _leaky(alpha)
    ex = jnp.where(mask, jnp.exp(alpha), 0.0)
    alpha_loop = _leaky(s_src + s_dst + jnp.dot(ea_mean, w_e))
    ex_loop = jnp.exp(alpha_loop)
    denom = jax.ops.segment_sum(ex, dst, num_segments=n) + ex_loop
    coef = ex / denom[dst]
    out = jax.ops.segment_sum(coef[:, None] * h[src], dst, num_segments=n)
    out = out + (ex_loop / denom)[:, None] * h
    return out + b


def kernel(x, edge_index, edge_attr, W_f, a_src_f, a_dst_f, W_e_f, a_e_f, b_f,
           W_b, a_src_b, a_dst_b, W_e_b, a_e_b, b_b, W_c, b_c, gamma, beta):
    src = edge_index[0]
    dst = edge_index[1]
    fmask = (edge_attr[:, 0] == 1) | (edge_attr[:, 2] == 1) | (edge_attr[:, 4] == 1)
    nl = src != dst
    fm = fmask & nl
    bm = (~fmask) & nl
    ea_mean_f = jnp.sum(jnp.where(fm[:, None], edge_attr, 0.0), axis=0) / jnp.sum(fm.astype(jnp.float32))
    ea_mean_b = jnp.sum(jnp.where(bm[:, None], edge_attr, 0.0), axis=0) / jnp.sum(bm.astype(jnp.float32))
    f_out = _gat_dir(x, src, dst, edge_attr, fm, ea_mean_f, W_f, a_src_f, a_dst_f, W_e_f, a_e_f, b_f)
    b_out = _gat_dir(x, src, dst, edge_attr, bm, ea_mean_b, W_b, a_src_b, a_dst_b, W_e_b, a_e_b, b_b)
    return _combine(f_out, b_out, W_c, b_c, gamma, beta)


# speedup vs baseline: 18.1605x; 13.1028x over previous
import jax
import jax.numpy as jnp
from jax import lax
from jax.experimental import pallas as pl
from jax.experimental.pallas import tpu as pltpu
from jax.experimental.pallas import tpu_sc as plsc

N = 10000
E = 320000
D = 128
C = 64
NP = 10240
NS = 16
NC = 2
ECH = 128
NCHG = E // ECH
CH_BASE = NCHG // NS
CH_REM = NCHG - CH_BASE * NS
NPT = NP // NS


def _sc_body(src_h, dst_h, e_h, ss_h, sd_h, h_h,
             agg_o, den_o,
             ss_v, sd_v, srcc_v, dstc_v, ec_v, ex_v, rows_v, zb_v,
             aggs, dens):
    cid = lax.axis_index("c")
    sid = lax.axis_index("s")

    zeros16 = jnp.zeros((16,), jnp.float32)
    for i in range(NPT // 16):
        zb_v[pl.ds(i * 16, 16)] = zeros16

    def _zero_row(r, _):
        for j in range(8):
            rows_v[r, pl.ds(j * 16, 16)] = zeros16
        return 0

    lax.fori_loop(0, ECH, _zero_row, 0)
    pltpu.sync_copy(zb_v, dens.at[pl.ds(sid * NPT, NPT)])
    for i in range(NP // ECH // NS):
        blk = (sid * (NP // ECH // NS) + i) * ECH
        pltpu.sync_copy(rows_v, aggs.at[pl.ds(blk, ECH)])

    pltpu.sync_copy(ss_h.at[cid], ss_v)
    pltpu.sync_copy(sd_h.at[cid], sd_v)
    plsc.subcore_barrier()

    gbase = sid * CH_BASE + jnp.minimum(sid, CH_REM)
    nch = CH_BASE + jnp.where(sid < CH_REM, 1, 0)

    def _load_chunk_and_ex(ci):
        goff = (gbase + ci) * ECH
        pltpu.sync_copy(src_h.at[pl.ds(goff, ECH)], srcc_v)
        pltpu.sync_copy(dst_h.at[pl.ds(goff, ECH)], dstc_v)
        pltpu.sync_copy(e_h.at[cid, pl.ds(goff, ECH)], ec_v)
        for g in range(ECH // 16):
            sl = pl.ds(g * 16, 16)
            a = (plsc.load_gather(ss_v, [srcc_v[sl]])
                 + plsc.load_gather(sd_v, [dstc_v[sl]]) + ec_v[sl])
            a = jnp.maximum(a, 0.2 * a)
            ex_v[sl] = jnp.exp(a)

    def _p1_chunk(ci, _):
        _load_chunk_and_ex(ci)
        pltpu.sync_copy(ex_v, dens.at[dstc_v], add=True)
        return 0

    lax.fori_loop(0, nch, _p1_chunk, 0)
    plsc.subcore_barrier()

    @pl.when(cid == 0)
    def _():
        pltpu.sync_copy(dens.at[pl.ds(sid * NPT, NPT)],
                        den_o.at[0, pl.ds(sid * NPT, NPT)])

    @pl.when(cid == 1)
    def _():
        pltpu.sync_copy(dens.at[pl.ds(sid * NPT, NPT)],
                        den_o.at[1, pl.ds(sid * NPT, NPT)])

    def _p2_chunk(ci, _):
        _load_chunk_and_ex(ci)
        pltpu.sync_copy(h_h.at[cid].at[srcc_v], rows_v)

        def _scale_row(k, _):
            cf = plsc.load_gather(ex_v, [jnp.full((16,), k, jnp.int32)])
            for j in range(8):
                sl = pl.ds(j * 16, 16)
                rows_v[k, sl] = rows_v[k, sl] * cf
            return 0

        lax.fori_loop(0, ECH, _scale_row, 0)
        pltpu.sync_copy(rows_v, aggs.at[dstc_v], add=True)
        return 0

    lax.fori_loop(0, nch, _p2_chunk, 0)
    plsc.subcore_barrier()

    @pl.when(cid == 0)
    def _():
        pltpu.sync_copy(aggs.at[pl.ds(sid * NPT, NPT)],
                        agg_o.at[0, pl.ds(sid * NPT, NPT)])

    @pl.when(cid == 1)
    def _():
        pltpu.sync_copy(aggs.at[pl.ds(sid * NPT, NPT)],
                        agg_o.at[1, pl.ds(sid * NPT, NPT)])


def _make_sc_fn():
    return pl.kernel(
        _sc_body,
        out_type=(jax.ShapeDtypeStruct((NC, NP, 128), jnp.float32),
                  jax.ShapeDtypeStruct((2, NP), jnp.float32)),
        mesh=plsc.VectorSubcoreMesh(core_axis_name="c", subcore_axis_name="s"),
        compiler_params=pltpu.CompilerParams(needs_layout_passes=False),
        scratch_types=[
            pltpu.VMEM((N,), jnp.float32),
            pltpu.VMEM((N,), jnp.float32),
            pltpu.VMEM((ECH,), jnp.int32),
            pltpu.VMEM((ECH,), jnp.int32),
            pltpu.VMEM((ECH,), jnp.float32),
            pltpu.VMEM((ECH,), jnp.float32),
            pltpu.VMEM((ECH, 128), jnp.float32),
            pltpu.VMEM((NPT,), jnp.float32),
            pltpu.VMEM_SHARED((NP, 128), jnp.float32),
            pltpu.VMEM_SHARED((NP,), jnp.float32),
        ],
    )


def _combine_body(aggf_ref, aggb_ref, hf_ref, hb_ref, exlf_ref, exlb_ref,
                  rdf_ref, rdb_ref, bf_ref, bb_ref,
                  Wc_ref, bc_ref, gamma_ref, beta_ref, out_ref):
    exlf = exlf_ref[...][:, None]
    exlb = exlb_ref[...][:, None]
    f_out = (aggf_ref[...] + exlf * hf_ref[...]) * rdf_ref[...][:, None] + bf_ref[...][None, :]
    b_out = (aggb_ref[...] + exlb * hb_ref[...]) * rdb_ref[...][:, None] + bb_ref[...][None, :]
    comb = jnp.concatenate([f_out, b_out], axis=1)
    z = lax.dot_general(comb, Wc_ref[...], (((1,), (1,)), ((), ())),
                        preferred_element_type=jnp.float32)
    z = z + bc_ref[...][None, :]
    mu = jnp.mean(z, axis=0, keepdims=True)
    var = jnp.mean((z - mu) ** 2, axis=0, keepdims=True)
    out = gamma_ref[...][None, :] * (z - mu) * lax.rsqrt(var + 1e-5) + beta_ref[...][None, :]
    out_ref[...] = jnp.maximum(out, 0.0)


def _leaky(v):
    return jnp.maximum(v, 0.2 * v)


def kernel(x, edge_index, edge_attr, W_f, a_src_f, a_dst_f, W_e_f, a_e_f, b_f,
           W_b, a_src_b, a_dst_b, W_e_b, a_e_b, b_b, W_c, b_c, gamma, beta):
    src = edge_index[0]
    dst = edge_index[1]
    fmask = (edge_attr[:, 0] == 1) | (edge_attr[:, 2] == 1) | (edge_attr[:, 4] == 1)
    nl = src != dst
    fm = fmask & nl
    bm = (~fmask) & nl
    ea_mean_f = jnp.sum(jnp.where(fm[:, None], edge_attr, 0.0), axis=0) / jnp.sum(fm.astype(jnp.float32))
    ea_mean_b = jnp.sum(jnp.where(bm[:, None], edge_attr, 0.0), axis=0) / jnp.sum(bm.astype(jnp.float32))

    h_f = x @ W_f
    h_b = x @ W_b
    ssf = h_f @ a_src_f
    sdf = h_f @ a_dst_f
    ssb = h_b @ a_src_b
    sdb = h_b @ a_dst_b
    wef = W_e_f @ a_e_f
    web = W_e_b @ a_e_b
    ef = jnp.where(fm, edge_attr @ wef, -1e30)
    eb = jnp.where(bm, edge_attr @ web, -1e30)

    ss = jnp.stack([ssf, ssb])
    sd = jnp.stack([sdf, sdb])
    e2 = jnp.stack([ef, eb])
    zpad = jnp.zeros((2, N, 128 - C), jnp.float32)
    h2 = jnp.concatenate([jnp.stack([h_f, h_b]), zpad], axis=2)

    agg, den = _make_sc_fn()(src, dst, e2, ss, sd, h2)

    ex_loop_f = jnp.exp(_leaky(ssf + sdf + jnp.dot(ea_mean_f, wef)))
    ex_loop_b = jnp.exp(_leaky(ssb + sdb + jnp.dot(ea_mean_b, web)))
    rdf = 1.0 / (den[0, :N] + ex_loop_f)
    rdb = 1.0 / (den[1, :N] + ex_loop_b)

    return pl.pallas_call(
        _combine_body,
        out_shape=jax.ShapeDtypeStruct((N, 128), jnp.float32),
    )(agg[0, :N, :C], agg[1, :N, :C], h_f, h_b, ex_loop_f, ex_loop_b,
      rdf, rdb, b_f, b_b, W_c, b_c, gamma, beta)


# speedup vs baseline: 29.9559x; 1.6495x over previous
import jax
import jax.numpy as jnp
from jax import lax
from jax.experimental import pallas as pl
from jax.experimental.pallas import tpu as pltpu
from jax.experimental.pallas import tpu_sc as plsc

N = 10000
E = 320000
D = 128
C = 64
NP = 10240
NS = 16
NC = 2
ECH = 128
NCHG = E // ECH
CH_BASE = NCHG // NS
CH_REM = NCHG - CH_BASE * NS
NPT = NP // NS


def _sc_body(epk_h, ss_h, sd_h, h_h,
             agg_o, den_o,
             ss_v, sd_v, epk_v, ex_v, rows_v, zb_v,
             aggs, dens):
    cid = lax.axis_index("c")
    sid = lax.axis_index("s")

    zeros16 = jnp.zeros((16,), jnp.float32)
    for i in range(NPT // 16):
        zb_v[pl.ds(i * 16, 16)] = zeros16

    def _zero_row(r, _):
        for j in range(C // 16):
            rows_v[r, pl.ds(j * 16, 16)] = zeros16
        return 0

    lax.fori_loop(0, ECH, _zero_row, 0)
    pltpu.sync_copy(zb_v, dens.at[pl.ds(sid * NPT, NPT)])
    for i in range(NP // ECH // NS):
        blk = (sid * (NP // ECH // NS) + i) * ECH
        pltpu.sync_copy(rows_v, aggs.at[pl.ds(blk, ECH)])

    pltpu.sync_copy(ss_h.at[cid], ss_v)
    pltpu.sync_copy(sd_h.at[cid], sd_v)
    plsc.subcore_barrier()

    gbase = sid * CH_BASE + jnp.minimum(sid, CH_REM)
    nch = CH_BASE + jnp.where(sid < CH_REM, 1, 0)

    def _chunk(ci, _):
        pltpu.sync_copy(epk_h.at[cid, gbase + ci], epk_v)
        for g in range(ECH // 16):
            sl = pl.ds(g * 16, 16)
            a = (plsc.load_gather(ss_v, [epk_v[0, sl]])
                 + plsc.load_gather(sd_v, [epk_v[1, sl]])
                 + plsc.bitcast(epk_v[2, sl], jnp.float32))
            a = jnp.maximum(a, 0.2 * a)
            ex_v[sl] = jnp.exp(a)
        pltpu.sync_copy(ex_v, dens.at[epk_v.at[1]], add=True)
        pltpu.sync_copy(h_h.at[cid].at[epk_v.at[0]], rows_v)

        def _scale_row(k, _):
            cf = plsc.load_gather(ex_v, [jnp.full((16,), k, jnp.int32)])
            for j in range(C // 16):
                sl = pl.ds(j * 16, 16)
                rows_v[k, sl] = rows_v[k, sl] * cf
            return 0

        lax.fori_loop(0, ECH, _scale_row, 0)
        pltpu.sync_copy(rows_v, aggs.at[epk_v.at[1]], add=True)
        return 0

    lax.fori_loop(0, nch, _chunk, 0)
    plsc.subcore_barrier()

    @pl.when(cid == 0)
    def _():
        pltpu.sync_copy(dens.at[pl.ds(sid * NPT, NPT)],
                        den_o.at[0, pl.ds(sid * NPT, NPT)])
        pltpu.sync_copy(aggs.at[pl.ds(sid * NPT, NPT)],
                        agg_o.at[0, pl.ds(sid * NPT, NPT)])

    @pl.when(cid == 1)
    def _():
        pltpu.sync_copy(dens.at[pl.ds(sid * NPT, NPT)],
                        den_o.at[1, pl.ds(sid * NPT, NPT)])
        pltpu.sync_copy(aggs.at[pl.ds(sid * NPT, NPT)],
                        agg_o.at[1, pl.ds(sid * NPT, NPT)])


def _make_sc_fn():
    return pl.kernel(
        _sc_body,
        out_type=(jax.ShapeDtypeStruct((NC, NP, C), jnp.float32),
                  jax.ShapeDtypeStruct((2, NP), jnp.float32)),
        mesh=plsc.VectorSubcoreMesh(core_axis_name="c", subcore_axis_name="s"),
        compiler_params=pltpu.CompilerParams(needs_layout_passes=False,
                                             use_tc_tiling_on_sc=False),
        scratch_types=[
            pltpu.VMEM((N,), jnp.float32),
            pltpu.VMEM((N,), jnp.float32),
            pltpu.VMEM((3, ECH), jnp.int32),
            pltpu.VMEM((ECH,), jnp.float32),
            pltpu.VMEM((ECH, C), jnp.float32),
            pltpu.VMEM((NPT,), jnp.float32),
            pltpu.VMEM_SHARED((NP, C), jnp.float32),
            pltpu.VMEM_SHARED((NP,), jnp.float32),
        ],
    )


def _combine_body(aggf_ref, aggb_ref, hf_ref, hb_ref, exlf_ref, exlb_ref,
                  rdf_ref, rdb_ref, bf_ref, bb_ref,
                  Wc_ref, bc_ref, gamma_ref, beta_ref, out_ref):
    exlf = exlf_ref[...][:, None]
    exlb = exlb_ref[...][:, None]
    f_out = (aggf_ref[...] + exlf * hf_ref[...]) * rdf_ref[...][:, None] + bf_ref[...][None, :]
    b_out = (aggb_ref[...] + exlb * hb_ref[...]) * rdb_ref[...][:, None] + bb_ref[...][None, :]
    comb = jnp.concatenate([f_out, b_out], axis=1)
    z = lax.dot_general(comb, Wc_ref[...], (((1,), (1,)), ((), ())),
                        preferred_element_type=jnp.float32)
    z = z + bc_ref[...][None, :]
    mu = jnp.mean(z, axis=0, keepdims=True)
    var = jnp.mean((z - mu) ** 2, axis=0, keepdims=True)
    out = gamma_ref[...][None, :] * (z - mu) * lax.rsqrt(var + 1e-5) + beta_ref[...][None, :]
    out_ref[...] = jnp.maximum(out, 0.0)


def _leaky(v):
    return jnp.maximum(v, 0.2 * v)


def kernel(x, edge_index, edge_attr, W_f, a_src_f, a_dst_f, W_e_f, a_e_f, b_f,
           W_b, a_src_b, a_dst_b, W_e_b, a_e_b, b_b, W_c, b_c, gamma, beta):
    src = edge_index[0]
    dst = edge_index[1]
    fmask = (edge_attr[:, 0] == 1) | (edge_attr[:, 2] == 1) | (edge_attr[:, 4] == 1)
    nl = src != dst
    fm = fmask & nl
    bm = (~fmask) & nl
    ea_mean_f = jnp.sum(jnp.where(fm[:, None], edge_attr, 0.0), axis=0) / jnp.sum(fm.astype(jnp.float32))
    ea_mean_b = jnp.sum(jnp.where(bm[:, None], edge_attr, 0.0), axis=0) / jnp.sum(bm.astype(jnp.float32))

    h_f = x @ W_f
    h_b = x @ W_b
    ssf = h_f @ a_src_f
    sdf = h_f @ a_dst_f
    ssb = h_b @ a_src_b
    sdb = h_b @ a_dst_b
    wef = W_e_f @ a_e_f
    web = W_e_b @ a_e_b
    ef = jnp.where(fm, edge_attr @ wef, -1e30)
    eb = jnp.where(bm, edge_attr @ web, -1e30)

    ss = jnp.stack([ssf, ssb])
    sd = jnp.stack([sdf, sdb])
    h2 = jnp.stack([h_f, h_b])

    src_r = jnp.broadcast_to(src.reshape(NCHG, ECH), (NC, NCHG, ECH))
    dst_r = jnp.broadcast_to(dst.reshape(NCHG, ECH), (NC, NCHG, ECH))
    e_r = lax.bitcast_convert_type(jnp.stack([ef, eb]).reshape(NC, NCHG, ECH),
                                   jnp.int32)
    epk = jnp.stack([src_r, dst_r, e_r], axis=2)

    agg, den = _make_sc_fn()(epk, ss, sd, h2)

    ex_loop_f = jnp.exp(_leaky(ssf + sdf + jnp.dot(ea_mean_f, wef)))
    ex_loop_b = jnp.exp(_leaky(ssb + sdb + jnp.dot(ea_mean_b, web)))
    rdf = 1.0 / (den[0, :N] + ex_loop_f)
    rdb = 1.0 / (den[1, :N] + ex_loop_b)

    return pl.pallas_call(
        _combine_body,
        out_shape=jax.ShapeDtypeStruct((N, 128), jnp.float32),
    )(agg[0, :N], agg[1, :N], h_f, h_b, ex_loop_f, ex_loop_b,
      rdf, rdb, b_f, b_b, W_c, b_c, gamma, beta)


# speedup vs baseline: 39.2697x; 1.3109x over previous
import jax
import jax.numpy as jnp
from jax import lax
from jax.experimental import pallas as pl
from jax.experimental.pallas import tpu as pltpu
from jax.experimental.pallas import tpu_sc as plsc

N = 10000
E = 320000
D = 128
C = 64
NP = 10240
NS = 16
NC = 2
ECH = 128
NCHG = E // ECH
CH_BASE = NCHG // NS
CH_REM = NCHG - CH_BASE * NS
NPT = NP // NS


def _sc_body(epk_h, ss_h, sd_h, h_h,
             agg_o, den_o,
             ss_v, sd_v, epk_v, ex_v, rows_v, zb_v,
             sem_lin, sem_gat, sem_sd, sem_sr,
             aggs, dens):
    cid = lax.axis_index("c")
    sid = lax.axis_index("s")

    zeros16 = jnp.zeros((16,), jnp.float32)
    for i in range(NPT // 16):
        zb_v[pl.ds(i * 16, 16)] = zeros16

    def _zero_row(r, _):
        for j in range(C // 16):
            rows_v[0, r, pl.ds(j * 16, 16)] = zeros16
        return 0

    lax.fori_loop(0, ECH, _zero_row, 0)
    pltpu.sync_copy(zb_v, dens.at[pl.ds(sid * NPT, NPT)])
    for i in range(NP // ECH // NS):
        blk = (sid * (NP // ECH // NS) + i) * ECH
        pltpu.sync_copy(rows_v.at[0], aggs.at[pl.ds(blk, ECH)])

    pltpu.sync_copy(ss_h.at[cid], ss_v)
    pltpu.sync_copy(sd_h.at[cid], sd_v)
    plsc.subcore_barrier()

    gbase = sid * CH_BASE + jnp.minimum(sid, CH_REM)
    nch = CH_BASE + jnp.where(sid < CH_REM, 1, 0)

    def _chunk_work(cc, b):
        b1 = (b + 1) % 3
        ep = epk_v.at[b]
        pltpu.make_async_copy(epk_h.at[cid, gbase + cc], ep,
                              sem_lin.at[b]).wait()

        @pl.when(cc >= 2)
        def _():
            pltpu.make_async_copy(ex_v.at[b1], dens.at[epk_v.at[b1].at[1]],
                                  sem_sd.at[b1]).wait()
            pltpu.make_async_copy(rows_v.at[b1], aggs.at[epk_v.at[b1].at[1]],
                                  sem_sr.at[b1]).wait()

        @pl.when(cc + 1 < nch)
        def _():
            pltpu.async_copy(epk_h.at[cid, gbase + cc + 1], epk_v.at[b1],
                             sem_lin.at[b1])

        gat = pltpu.async_copy(h_h.at[cid].at[ep.at[0]], rows_v.at[b], sem_gat)

        for g in range(ECH // 16):
            sl = pl.ds(g * 16, 16)
            a = (plsc.load_gather(ss_v, [epk_v[b, 0, sl]])
                 + plsc.load_gather(sd_v, [epk_v[b, 1, sl]])
                 + plsc.bitcast(epk_v[b, 2, sl], jnp.float32))
            a = jnp.maximum(a, 0.2 * a)
            ex_v[b, sl] = jnp.exp(a)
        pltpu.async_copy(ex_v.at[b], dens.at[ep.at[1]], sem_sd.at[b], add=True)

        gat.wait()

        def _scale_row(k, _):
            cf = plsc.load_gather(ex_v.at[b], [jnp.full((16,), k, jnp.int32)])
            for j in range(C // 16):
                sl = pl.ds(j * 16, 16)
                rows_v[b, k, sl] = rows_v[b, k, sl] * cf
            return 0

        lax.fori_loop(0, ECH, _scale_row, 0)
        pltpu.async_copy(rows_v.at[b], aggs.at[ep.at[1]], sem_sr.at[b],
                         add=True)

    pltpu.async_copy(epk_h.at[cid, gbase], epk_v.at[0], sem_lin.at[0])

    def _body(ci, _):
        for b in range(3):
            cc = ci * 3 + b
            pl.when(cc < nch)(lambda cc=cc, b=b: _chunk_work(cc, b))
        return 0

    lax.fori_loop(0, (nch + 2) // 3, _body, 0)

    for b in range(3):
        cond = ((nch - 2) % 3 == b) | ((nch - 1) % 3 == b)

        @pl.when(cond)
        def _(b=b):
            pltpu.make_async_copy(ex_v.at[b], dens.at[epk_v.at[b].at[1]],
                                  sem_sd.at[b]).wait()
            pltpu.make_async_copy(rows_v.at[b], aggs.at[epk_v.at[b].at[1]],
                                  sem_sr.at[b]).wait()
    plsc.subcore_barrier()

    @pl.when(cid == 0)
    def _():
        pltpu.sync_copy(dens.at[pl.ds(sid * NPT, NPT)],
                        den_o.at[0, pl.ds(sid * NPT, NPT)])
        pltpu.sync_copy(aggs.at[pl.ds(sid * NPT, NPT)],
                        agg_o.at[0, pl.ds(sid * NPT, NPT)])

    @pl.when(cid == 1)
    def _():
        pltpu.sync_copy(dens.at[pl.ds(sid * NPT, NPT)],
                        den_o.at[1, pl.ds(sid * NPT, NPT)])
        pltpu.sync_copy(aggs.at[pl.ds(sid * NPT, NPT)],
                        agg_o.at[1, pl.ds(sid * NPT, NPT)])


def _make_sc_fn():
    return pl.kernel(
        _sc_body,
        out_type=(jax.ShapeDtypeStruct((NC, NP, C), jnp.float32),
                  jax.ShapeDtypeStruct((2, NP), jnp.float32)),
        mesh=plsc.VectorSubcoreMesh(core_axis_name="c", subcore_axis_name="s"),
        compiler_params=pltpu.CompilerParams(needs_layout_passes=False,
                                             use_tc_tiling_on_sc=False),
        scratch_types=[
            pltpu.VMEM((N,), jnp.float32),
            pltpu.VMEM((N,), jnp.float32),
            pltpu.VMEM((3, 3, ECH), jnp.int32),
            pltpu.VMEM((3, ECH), jnp.float32),
            pltpu.VMEM((3, ECH, C), jnp.float32),
            pltpu.VMEM((NPT,), jnp.float32),
            pltpu.SemaphoreType.DMA((3,)),
            pltpu.SemaphoreType.DMA,
            pltpu.SemaphoreType.DMA((3,)),
            pltpu.SemaphoreType.DMA((3,)),
            pltpu.VMEM_SHARED((NP, C), jnp.float32),
            pltpu.VMEM_SHARED((NP,), jnp.float32),
        ],
    )


def _combine_body(aggf_ref, aggb_ref, hf_ref, hb_ref, exlf_ref, exlb_ref,
                  rdf_ref, rdb_ref, bf_ref, bb_ref,
                  Wc_ref, bc_ref, gamma_ref, beta_ref, out_ref):
    exlf = exlf_ref[...][:, None]
    exlb = exlb_ref[...][:, None]
    f_out = (aggf_ref[...] + exlf * hf_ref[...]) * rdf_ref[...][:, None] + bf_ref[...][None, :]
    b_out = (aggb_ref[...] + exlb * hb_ref[...]) * rdb_ref[...][:, None] + bb_ref[...][None, :]
    comb = jnp.concatenate([f_out, b_out], axis=1)
    z = lax.dot_general(comb, Wc_ref[...], (((1,), (1,)), ((), ())),
                        preferred_element_type=jnp.float32)
    z = z + bc_ref[...][None, :]
    mu = jnp.mean(z, axis=0, keepdims=True)
    var = jnp.mean((z - mu) ** 2, axis=0, keepdims=True)
    out = gamma_ref[...][None, :] * (z - mu) * lax.rsqrt(var + 1e-5) + beta_ref[...][None, :]
    out_ref[...] = jnp.maximum(out, 0.0)


def _leaky(v):
    return jnp.maximum(v, 0.2 * v)


def kernel(x, edge_index, edge_attr, W_f, a_src_f, a_dst_f, W_e_f, a_e_f, b_f,
           W_b, a_src_b, a_dst_b, W_e_b, a_e_b, b_b, W_c, b_c, gamma, beta):
    src = edge_index[0]
    dst = edge_index[1]
    fmask = (edge_attr[:, 0] == 1) | (edge_attr[:, 2] == 1) | (edge_attr[:, 4] == 1)
    nl = src != dst
    fm = fmask & nl
    bm = (~fmask) & nl
    ea_mean_f = jnp.sum(jnp.where(fm[:, None], edge_attr, 0.0), axis=0) / jnp.sum(fm.astype(jnp.float32))
    ea_mean_b = jnp.sum(jnp.where(bm[:, None], edge_attr, 0.0), axis=0) / jnp.sum(bm.astype(jnp.float32))

    h_f = x @ W_f
    h_b = x @ W_b
    ssf = h_f @ a_src_f
    sdf = h_f @ a_dst_f
    ssb = h_b @ a_src_b
    sdb = h_b @ a_dst_b
    wef = W_e_f @ a_e_f
    web = W_e_b @ a_e_b
    ef = jnp.where(fm, edge_attr @ wef, -1e30)
    eb = jnp.where(bm, edge_attr @ web, -1e30)

    ss = jnp.stack([ssf, ssb])
    sd = jnp.stack([sdf, sdb])
    h2 = jnp.stack([h_f, h_b])

    src_r = jnp.broadcast_to(src.reshape(NCHG, ECH), (NC, NCHG, ECH))
    dst_r = jnp.broadcast_to(dst.reshape(NCHG, ECH), (NC, NCHG, ECH))
    e_r = lax.bitcast_convert_type(jnp.stack([ef, eb]).reshape(NC, NCHG, ECH),
                                   jnp.int32)
    epk = jnp.stack([src_r, dst_r, e_r], axis=2)

    agg, den = _make_sc_fn()(epk, ss, sd, h2)

    ex_loop_f = jnp.exp(_leaky(ssf + sdf + jnp.dot(ea_mean_f, wef)))
    ex_loop_b = jnp.exp(_leaky(ssb + sdb + jnp.dot(ea_mean_b, web)))
    rdf = 1.0 / (den[0, :N] + ex_loop_f)
    rdb = 1.0 / (den[1, :N] + ex_loop_b)

    return pl.pallas_call(
        _combine_body,
        out_shape=jax.ShapeDtypeStruct((N, 128), jnp.float32),
    )(agg[0, :N], agg[1, :N], h_f, h_b, ex_loop_f, ex_loop_b,
      rdf, rdb, b_f, b_b, W_c, b_c, gamma, beta)


# speedup vs baseline: 43.0811x; 1.0971x over previous
import jax
import jax.numpy as jnp
from jax import lax
from jax.experimental import pallas as pl
from jax.experimental.pallas import tpu as pltpu
from jax.experimental.pallas import tpu_sc as plsc

N = 10000
E = 320000
D = 128
C = 64
NP = 10240
NS = 16
NC = 2
ECH = 128
NCHG = E // ECH
CH_BASE = NCHG // NS
CH_REM = NCHG - CH_BASE * NS
NPT = NP // NS
W = C + 16


def _sc_body(epk_h, ss_h, sd_h, h_h,
             agg_o,
             ss_v, sd_v, epk_v, ex_v, rows_v,
             sem_lin, sem_gat, sem_sr,
             aggs):
    cid = lax.axis_index("c")
    sid = lax.axis_index("s")

    zeros16 = jnp.zeros((16,), jnp.float32)

    def _zero_row(r, _):
        for j in range(W // 16):
            rows_v[0, r, pl.ds(j * 16, 16)] = zeros16
        return 0

    lax.fori_loop(0, ECH, _zero_row, 0)
    for i in range(NP // ECH // NS):
        blk = (sid * (NP // ECH // NS) + i) * ECH
        pltpu.sync_copy(rows_v.at[0], aggs.at[pl.ds(blk, ECH)])

    pltpu.sync_copy(ss_h.at[cid], ss_v)
    pltpu.sync_copy(sd_h.at[cid], sd_v)
    plsc.subcore_barrier()

    gbase = sid * CH_BASE + jnp.minimum(sid, CH_REM)
    nch = CH_BASE + jnp.where(sid < CH_REM, 1, 0)
    lane0 = lax.iota(jnp.int32, 16) == 0

    def _scale_and_scatter(cc, b):
        pltpu.make_async_copy(h_h.at[cid].at[epk_v.at[b].at[0]],
                              rows_v.at[b], sem_gat.at[b]).wait()

        def _scale_row(k, _):
            cf = plsc.load_gather(ex_v.at[b], [jnp.full((16,), k, jnp.int32)])
            for j in range(C // 16):
                sl = pl.ds(j * 16, 16)
                rows_v[b, k, sl] = rows_v[b, k, sl] * cf
            rows_v[b, k, pl.ds(C, 16)] = jnp.where(lane0, cf, 0.0)
            return 0

        lax.fori_loop(0, ECH, _scale_row, 0)
        pltpu.async_copy(rows_v.at[b], aggs.at[epk_v.at[b].at[1]],
                         sem_sr.at[b], add=True)

    def _chunk_work(cc, b):
        b1 = (b + 1) % 3
        bp = (b + 2) % 3
        ep = epk_v.at[b]
        pltpu.make_async_copy(epk_h.at[cid, gbase + cc], ep,
                              sem_lin.at[b]).wait()

        @pl.when(cc >= 2)
        def _():
            pltpu.make_async_copy(rows_v.at[b1], aggs.at[epk_v.at[b1].at[1]],
                                  sem_sr.at[b1]).wait()

        @pl.when(cc + 1 < nch)
        def _():
            pltpu.async_copy(epk_h.at[cid, gbase + cc + 1], epk_v.at[b1],
                             sem_lin.at[b1])

        pltpu.async_copy(h_h.at[cid].at[ep.at[0]], rows_v.at[b],
                         sem_gat.at[b])

        for g in range(ECH // 16):
            sl = pl.ds(g * 16, 16)
            a = (plsc.load_gather(ss_v, [epk_v[b, 0, sl]])
                 + plsc.load_gather(sd_v, [epk_v[b, 1, sl]])
                 + plsc.bitcast(epk_v[b, 2, sl], jnp.float32))
            a = jnp.maximum(a, 0.2 * a)
            ex_v[b, sl] = jnp.exp(a)

        pl.when(cc >= 1)(lambda: _scale_and_scatter(cc - 1, bp))

    pltpu.async_copy(epk_h.at[cid, gbase], epk_v.at[0], sem_lin.at[0])

    def _body(ci, _):
        for b in range(3):
            cc = ci * 3 + b
            pl.when(cc < nch)(lambda cc=cc, b=b: _chunk_work(cc, b))
        return 0

    lax.fori_loop(0, (nch + 2) // 3, _body, 0)

    for b in range(3):
        pl.when((nch - 1) % 3 == b)(
            lambda b=b: _scale_and_scatter(nch - 1, b))

    for b in range(3):
        cond = ((nch - 2) % 3 == b) | ((nch - 1) % 3 == b)

        @pl.when(cond)
        def _(b=b):
            pltpu.make_async_copy(rows_v.at[b], aggs.at[epk_v.at[b].at[1]],
                                  sem_sr.at[b]).wait()
    plsc.subcore_barrier()

    @pl.when(cid == 0)
    def _():
        pltpu.sync_copy(aggs.at[pl.ds(sid * NPT, NPT)],
                        agg_o.at[0, pl.ds(sid * NPT, NPT)])

    @pl.when(cid == 1)
    def _():
        pltpu.sync_copy(aggs.at[pl.ds(sid * NPT, NPT)],
                        agg_o.at[1, pl.ds(sid * NPT, NPT)])


def _make_sc_fn():
    return pl.kernel(
        _sc_body,
        out_type=jax.ShapeDtypeStruct((NC, NP, W), jnp.float32),
        mesh=plsc.VectorSubcoreMesh(core_axis_name="c", subcore_axis_name="s"),
        compiler_params=pltpu.CompilerParams(needs_layout_passes=False,
                                             use_tc_tiling_on_sc=False),
        scratch_types=[
            pltpu.VMEM((N,), jnp.float32),
            pltpu.VMEM((N,), jnp.float32),
            pltpu.VMEM((3, 3, ECH), jnp.int32),
            pltpu.VMEM((3, ECH), jnp.float32),
            pltpu.VMEM((3, ECH, W), jnp.float32),
            pltpu.SemaphoreType.DMA((3,)),
            pltpu.SemaphoreType.DMA((3,)),
            pltpu.SemaphoreType.DMA((3,)),
            pltpu.VMEM_SHARED((NP, W), jnp.float32),
        ],
    )


def _combine_body(aggf_ref, aggb_ref, hf_ref, hb_ref, exlf_ref, exlb_ref,
                  rdf_ref, rdb_ref, bf_ref, bb_ref,
                  Wc_ref, bc_ref, gamma_ref, beta_ref, out_ref):
    exlf = exlf_ref[...][:, None]
    exlb = exlb_ref[...][:, None]
    f_out = (aggf_ref[...] + exlf * hf_ref[...]) * rdf_ref[...][:, None] + bf_ref[...][None, :]
    b_out = (aggb_ref[...] + exlb * hb_ref[...]) * rdb_ref[...][:, None] + bb_ref[...][None, :]
    comb = jnp.concatenate([f_out, b_out], axis=1)
    z = lax.dot_general(comb, Wc_ref[...], (((1,), (1,)), ((), ())),
                        preferred_element_type=jnp.float32)
    z = z + bc_ref[...][None, :]
    mu = jnp.mean(z, axis=0, keepdims=True)
    var = jnp.mean((z - mu) ** 2, axis=0, keepdims=True)
    out = gamma_ref[...][None, :] * (z - mu) * lax.rsqrt(var + 1e-5) + beta_ref[...][None, :]
    out_ref[...] = jnp.maximum(out, 0.0)


def _leaky(v):
    return jnp.maximum(v, 0.2 * v)


def kernel(x, edge_index, edge_attr, W_f, a_src_f, a_dst_f, W_e_f, a_e_f, b_f,
           W_b, a_src_b, a_dst_b, W_e_b, a_e_b, b_b, W_c, b_c, gamma, beta):
    src = edge_index[0]
    dst = edge_index[1]
    fmask = (edge_attr[:, 0] == 1) | (edge_attr[:, 2] == 1) | (edge_attr[:, 4] == 1)
    nl = src != dst
    fm = fmask & nl
    bm = (~fmask) & nl
    ea_mean_f = jnp.sum(jnp.where(fm[:, None], edge_attr, 0.0), axis=0) / jnp.sum(fm.astype(jnp.float32))
    ea_mean_b = jnp.sum(jnp.where(bm[:, None], edge_attr, 0.0), axis=0) / jnp.sum(bm.astype(jnp.float32))

    h_f = x @ W_f
    h_b = x @ W_b
    ssf = h_f @ a_src_f
    sdf = h_f @ a_dst_f
    ssb = h_b @ a_src_b
    sdb = h_b @ a_dst_b
    wef = W_e_f @ a_e_f
    web = W_e_b @ a_e_b
    ef = jnp.where(fm, edge_attr @ wef, -1e30)
    eb = jnp.where(bm, edge_attr @ web, -1e30)

    ss = jnp.stack([ssf, ssb])
    sd = jnp.stack([sdf, sdb])
    h2 = jnp.concatenate([jnp.stack([h_f, h_b]),
                          jnp.zeros((NC, N, W - C), jnp.float32)], axis=2)

    src_r = jnp.broadcast_to(src.reshape(NCHG, ECH), (NC, NCHG, ECH))
    dst_r = jnp.broadcast_to(dst.reshape(NCHG, ECH), (NC, NCHG, ECH))
    e_r = lax.bitcast_convert_type(jnp.stack([ef, eb]).reshape(NC, NCHG, ECH),
                                   jnp.int32)
    epk = jnp.stack([src_r, dst_r, e_r], axis=2)

    agg = _make_sc_fn()(epk, ss, sd, h2)

    ex_loop_f = jnp.exp(_leaky(ssf + sdf + jnp.dot(ea_mean_f, wef)))
    ex_loop_b = jnp.exp(_leaky(ssb + sdb + jnp.dot(ea_mean_b, web)))
    rdf = 1.0 / (agg[0, :N, C] + ex_loop_f)
    rdb = 1.0 / (agg[1, :N, C] + ex_loop_b)

    return pl.pallas_call(
        _combine_body,
        out_shape=jax.ShapeDtypeStruct((N, 128), jnp.float32),
    )(agg[0, :N, :C], agg[1, :N, :C], h_f, h_b, ex_loop_f, ex_loop_b,
      rdf, rdb, b_f, b_b, W_c, b_c, gamma, beta)


# speedup vs baseline: 44.4080x; 1.0308x over previous
import jax
import jax.numpy as jnp
from jax import lax
from jax.experimental import pallas as pl
from jax.experimental.pallas import tpu as pltpu
from jax.experimental.pallas import tpu_sc as plsc

N = 10000
E = 320000
D = 128
C = 64
NP = 10240
NS = 16
NC = 2
ECH = 128
NCHG = E // ECH
CH_BASE = NCHG // NS
CH_REM = NCHG - CH_BASE * NS
NPT = NP // NS
W = C + 16


def _sc_body(epk_h, ss_h, sd_h, h_h,
             agg_o,
             ss_v, sd_v, epk_v, ex_v, rows_v,
             sem_lin, sem_gat, sem_sr,
             aggs):
    cid = lax.axis_index("c")
    sid = lax.axis_index("s")

    zeros16 = jnp.zeros((16,), jnp.float32)

    def _zero_row(r, _):
        for j in range(W // 16):
            rows_v[0, r, pl.ds(j * 16, 16)] = zeros16
        return 0

    lax.fori_loop(0, ECH, _zero_row, 0)
    for i in range(NP // ECH // NS):
        blk = (sid * (NP // ECH // NS) + i) * ECH
        pltpu.sync_copy(rows_v.at[0], aggs.at[pl.ds(blk, ECH)])

    pltpu.sync_copy(ss_h.at[cid], ss_v)
    pltpu.sync_copy(sd_h.at[cid], sd_v)
    plsc.subcore_barrier()

    gbase = sid * CH_BASE + jnp.minimum(sid, CH_REM)
    nch = CH_BASE + jnp.where(sid < CH_REM, 1, 0)
    lane0 = lax.iota(jnp.int32, 16) == 0

    def _scale_and_scatter(cc, b):
        pltpu.make_async_copy(h_h.at[cid].at[epk_v.at[b].at[0]],
                              rows_v.at[b], sem_gat.at[b]).wait()

        def _scale_row(k4, _):
            for dk in range(4):
                k = k4 * 4 + dk
                cf = plsc.load_gather(ex_v.at[b],
                                      [jnp.full((16,), k, jnp.int32)])
                for j in range(C // 16):
                    sl = pl.ds(j * 16, 16)
                    rows_v[b, k, sl] = rows_v[b, k, sl] * cf
                rows_v[b, k, pl.ds(C, 16)] = jnp.where(lane0, cf, 0.0)
            return 0

        lax.fori_loop(0, ECH // 4, _scale_row, 0)
        pltpu.async_copy(rows_v.at[b], aggs.at[epk_v.at[b].at[1]],
                         sem_sr.at[b], add=True)

    def _chunk_work(cc, b):
        b1 = (b + 1) % 3
        bp = (b + 2) % 3
        ep = epk_v.at[b]
        pltpu.make_async_copy(epk_h.at[cid, gbase + cc], ep,
                              sem_lin.at[b]).wait()

        @pl.when(cc >= 2)
        def _():
            pltpu.make_async_copy(rows_v.at[b1], aggs.at[epk_v.at[b1].at[1]],
                                  sem_sr.at[b1]).wait()

        @pl.when(cc + 1 < nch)
        def _():
            pltpu.async_copy(epk_h.at[cid, gbase + cc + 1], epk_v.at[b1],
                             sem_lin.at[b1])

        pltpu.async_copy(h_h.at[cid].at[ep.at[0]], rows_v.at[b],
                         sem_gat.at[b])

        for g in range(ECH // 16):
            sl = pl.ds(g * 16, 16)
            a = (plsc.load_gather(ss_v, [epk_v[b, 0, sl]])
                 + plsc.load_gather(sd_v, [epk_v[b, 1, sl]])
                 + plsc.bitcast(epk_v[b, 2, sl], jnp.float32))
            a = jnp.maximum(a, 0.2 * a)
            ex_v[b, sl] = jnp.exp(a)

        pl.when(cc >= 1)(lambda: _scale_and_scatter(cc - 1, bp))

    pltpu.async_copy(epk_h.at[cid, gbase], epk_v.at[0], sem_lin.at[0])

    def _body(ci, _):
        for b in range(3):
            cc = ci * 3 + b
            pl.when(cc < nch)(lambda cc=cc, b=b: _chunk_work(cc, b))
        return 0

    lax.fori_loop(0, (nch + 2) // 3, _body, 0)

    for b in range(3):
        pl.when((nch - 1) % 3 == b)(
            lambda b=b: _scale_and_scatter(nch - 1, b))

    for b in range(3):
        cond = ((nch - 2) % 3 == b) | ((nch - 1) % 3 == b)

        @pl.when(cond)
        def _(b=b):
            pltpu.make_async_copy(rows_v.at[b], aggs.at[epk_v.at[b].at[1]],
                                  sem_sr.at[b]).wait()
    plsc.subcore_barrier()

    @pl.when(cid == 0)
    def _():
        pltpu.sync_copy(aggs.at[pl.ds(sid * NPT, NPT)],
                        agg_o.at[0, pl.ds(sid * NPT, NPT)])

    @pl.when(cid == 1)
    def _():
        pltpu.sync_copy(aggs.at[pl.ds(sid * NPT, NPT)],
                        agg_o.at[1, pl.ds(sid * NPT, NPT)])


def _make_sc_fn():
    return pl.kernel(
        _sc_body,
        out_type=jax.ShapeDtypeStruct((NC, NP, W), jnp.float32),
        mesh=plsc.VectorSubcoreMesh(core_axis_name="c", subcore_axis_name="s"),
        compiler_params=pltpu.CompilerParams(needs_layout_passes=False,
                                             use_tc_tiling_on_sc=False),
        scratch_types=[
            pltpu.VMEM((N,), jnp.float32),
            pltpu.VMEM((N,), jnp.float32),
            pltpu.VMEM((3, 3, ECH), jnp.int32),
            pltpu.VMEM((3, ECH), jnp.float32),
            pltpu.VMEM((3, ECH, W), jnp.float32),
            pltpu.SemaphoreType.DMA((3,)),
            pltpu.SemaphoreType.DMA((3,)),
            pltpu.SemaphoreType.DMA((3,)),
            pltpu.VMEM_SHARED((NP, W), jnp.float32),
        ],
    )


def _combine_body(aggf_ref, aggb_ref, hf_ref, hb_ref, exlf_ref, exlb_ref,
                  rdf_ref, rdb_ref, bf_ref, bb_ref,
                  Wc_ref, bc_ref, gamma_ref, beta_ref, out_ref):
    exlf = exlf_ref[...][:, None]
    exlb = exlb_ref[...][:, None]
    f_out = (aggf_ref[...] + exlf * hf_ref[...]) * rdf_ref[...][:, None] + bf_ref[...][None, :]
    b_out = (aggb_ref[...] + exlb * hb_ref[...]) * rdb_ref[...][:, None] + bb_ref[...][None, :]
    comb = jnp.concatenate([f_out, b_out], axis=1)
    z = lax.dot_general(comb, Wc_ref[...], (((1,), (1,)), ((), ())),
                        preferred_element_type=jnp.float32)
    z = z + bc_ref[...][None, :]
    mu = jnp.mean(z, axis=0, keepdims=True)
    var = jnp.mean((z - mu) ** 2, axis=0, keepdims=True)
    out = gamma_ref[...][None, :] * (z - mu) * lax.rsqrt(var + 1e-5) + beta_ref[...][None, :]
    out_ref[...] = jnp.maximum(out, 0.0)


def _leaky(v):
    return jnp.maximum(v, 0.2 * v)


def kernel(x, edge_index, edge_attr, W_f, a_src_f, a_dst_f, W_e_f, a_e_f, b_f,
           W_b, a_src_b, a_dst_b, W_e_b, a_e_b, b_b, W_c, b_c, gamma, beta):
    src = edge_index[0]
    dst = edge_index[1]
    fmask = (edge_attr[:, 0] == 1) | (edge_attr[:, 2] == 1) | (edge_attr[:, 4] == 1)
    nl = src != dst
    fm = fmask & nl
    bm = (~fmask) & nl
    ea_mean_f = jnp.sum(jnp.where(fm[:, None], edge_attr, 0.0), axis=0) / jnp.sum(fm.astype(jnp.float32))
    ea_mean_b = jnp.sum(jnp.where(bm[:, None], edge_attr, 0.0), axis=0) / jnp.sum(bm.astype(jnp.float32))

    h_f = x @ W_f
    h_b = x @ W_b
    ssf = h_f @ a_src_f
    sdf = h_f @ a_dst_f
    ssb = h_b @ a_src_b
    sdb = h_b @ a_dst_b
    wef = W_e_f @ a_e_f
    web = W_e_b @ a_e_b
    ef = jnp.where(fm, edge_attr @ wef, -1e30)
    eb = jnp.where(bm, edge_attr @ web, -1e30)

    ss = jnp.stack([ssf, ssb])
    sd = jnp.stack([sdf, sdb])
    h2 = jnp.concatenate([jnp.stack([h_f, h_b]),
                          jnp.zeros((NC, N, W - C), jnp.float32)], axis=2)

    src_r = jnp.broadcast_to(src.reshape(NCHG, ECH), (NC, NCHG, ECH))
    dst_r = jnp.broadcast_to(dst.reshape(NCHG, ECH), (NC, NCHG, ECH))
    e_r = lax.bitcast_convert_type(jnp.stack([ef, eb]).reshape(NC, NCHG, ECH),
                                   jnp.int32)
    epk = jnp.stack([src_r, dst_r, e_r], axis=2)

    agg = _make_sc_fn()(epk, ss, sd, h2)

    ex_loop_f = jnp.exp(_leaky(ssf + sdf + jnp.dot(ea_mean_f, wef)))
    ex_loop_b = jnp.exp(_leaky(ssb + sdb + jnp.dot(ea_mean_b, web)))
    rdf = 1.0 / (agg[0, :N, C] + ex_loop_f)
    rdb = 1.0 / (agg[1, :N, C] + ex_loop_b)

    return pl.pallas_call(
        _combine_body,
        out_shape=jax.ShapeDtypeStruct((N, 128), jnp.float32),
    )(agg[0, :N, :C], agg[1, :N, :C], h_f, h_b, ex_loop_f, ex_loop_b,
      rdf, rdb, b_f, b_b, W_c, b_c, gamma, beta)
